# initial kernel scaffold (unmeasured)
import jax
import jax.numpy as jnp
from jax import lax
from jax.experimental import pallas as pl
from jax.experimental.pallas import tpu as pltpu

N_DEV = 4


def kernel(x, w_mat):
    m, k = x.shape
    _, n = w_mat.shape
    chunk = m // N_DEV
    half = n // 2

    def body(x_ref, w_ref, out_ref, cw_ref, ccw_ref,
             cw_send, cw_recv, ccw_send, ccw_recv):
        d = lax.axis_index("i")
        right = lax.rem(d + 1, N_DEV)
        left = lax.rem(d + 3, N_DEV)

        barrier = pltpu.get_barrier_semaphore()
        for nbr in (left, right):
            pl.semaphore_signal(barrier, inc=1, device_id=(nbr,),
                                device_id_type=pl.DeviceIdType.MESH)
        pl.semaphore_wait(barrier, 2)

        for c in range(N_DEV):
            out_ref[pl.ds(c * chunk, chunk), :] = jnp.dot(
                x_ref[pl.ds(c * chunk, chunk), :], w_ref[...],
                preferred_element_type=jnp.float32)

        def cw_rows(idx):
            return pl.ds(idx * chunk, chunk)

        for s in range(N_DEV - 1):
            c_cw = lax.rem(d - s + N_DEV, N_DEV)
            c_ccw = lax.rem(d + s, N_DEV)
            if s > 0:
                out_ref[cw_rows(c_cw), pl.ds(0, half)] = (
                    out_ref[cw_rows(c_cw), pl.ds(0, half)] + cw_ref[s - 1])
                out_ref[cw_rows(c_ccw), pl.ds(half, half)] = (
                    out_ref[cw_rows(c_ccw), pl.ds(half, half)] + ccw_ref[s - 1])
            cw = pltpu.make_async_remote_copy(
                src_ref=out_ref.at[cw_rows(c_cw), pl.ds(0, half)],
                dst_ref=cw_ref.at[s],
                send_sem=cw_send.at[s], recv_sem=cw_recv.at[s],
                device_id=(right,), device_id_type=pl.DeviceIdType.MESH)
            ccw = pltpu.make_async_remote_copy(
                src_ref=out_ref.at[cw_rows(c_ccw), pl.ds(half, half)],
                dst_ref=ccw_ref.at[s],
                send_sem=ccw_send.at[s], recv_sem=ccw_recv.at[s],
                device_id=(left,), device_id_type=pl.DeviceIdType.MESH)
            cw.start()
            ccw.start()
            cw.wait()
            ccw.wait()

        oc_cw = lax.rem(d + 1, N_DEV)
        oc_ccw = lax.rem(d + 3, N_DEV)
        out_ref[cw_rows(oc_cw), pl.ds(0, half)] = jnp.maximum(
            out_ref[cw_rows(oc_cw), pl.ds(0, half)] + cw_ref[N_DEV - 2], 0.0)
        out_ref[cw_rows(oc_ccw), pl.ds(half, half)] = jnp.maximum(
            out_ref[cw_rows(oc_ccw), pl.ds(half, half)] + ccw_ref[N_DEV - 2],
            0.0)

        for t in range(N_DEV - 1):
            slot = (N_DEV - 1) + t
            g_cw = lax.rem(d + 1 - t + N_DEV, N_DEV)
            g_ccw = lax.rem(d + 3 + t, N_DEV)
            cw = pltpu.make_async_remote_copy(
                src_ref=out_ref.at[cw_rows(g_cw), pl.ds(0, half)],
                dst_ref=cw_ref.at[slot],
                send_sem=cw_send.at[slot], recv_sem=cw_recv.at[slot],
                device_id=(right,), device_id_type=pl.DeviceIdType.MESH)
            ccw = pltpu.make_async_remote_copy(
                src_ref=out_ref.at[cw_rows(g_ccw), pl.ds(half, half)],
                dst_ref=ccw_ref.at[slot],
                send_sem=ccw_send.at[slot], recv_sem=ccw_recv.at[slot],
                device_id=(left,), device_id_type=pl.DeviceIdType.MESH)
            cw.start()
            ccw.start()
            cw.wait()
            ccw.wait()
            r_cw = lax.rem(d - t + N_DEV, N_DEV)
            r_ccw = lax.rem(d + t, N_DEV)
            out_ref[cw_rows(r_cw), pl.ds(0, half)] = cw_ref[slot]
            out_ref[cw_rows(r_ccw), pl.ds(half, half)] = ccw_ref[slot]

    n_slots = 2 * (N_DEV - 1)
    return pl.pallas_call(
        body,
        out_shape=jax.ShapeDtypeStruct((m, n), jnp.float32),
        in_specs=[pl.BlockSpec(memory_space=pltpu.VMEM),
                  pl.BlockSpec(memory_space=pltpu.VMEM)],
        out_specs=pl.BlockSpec(memory_space=pltpu.VMEM),
        scratch_shapes=[
            pltpu.VMEM((n_slots, chunk, half), jnp.float32),
            pltpu.VMEM((n_slots, chunk, half), jnp.float32),
            pltpu.SemaphoreType.DMA((n_slots,)),
            pltpu.SemaphoreType.DMA((n_slots,)),
            pltpu.SemaphoreType.DMA((n_slots,)),
            pltpu.SemaphoreType.DMA((n_slots,)),
        ],
        compiler_params=pltpu.CompilerParams(collective_id=0),
    )(x, w_mat)


# baseline (device time: 176837 ns/iter reference)
import jax
import jax.numpy as jnp
from jax import lax
from jax.experimental import pallas as pl
from jax.experimental.pallas import tpu as pltpu

N_DEV = 4


def kernel(x, w_mat):
    m, k = x.shape
    _, n = w_mat.shape
    chunk = m // N_DEV
    half = n // 2

    def body(x_ref, w_ref, out_ref, cw_ref, ccw_ref,
             cw_send, cw_recv, ccw_send, ccw_recv):
        d = lax.axis_index("i")
        right = lax.rem(d + 1, N_DEV)
        left = lax.rem(d + 3, N_DEV)

        barrier = pltpu.get_barrier_semaphore()
        for nbr in (left, right):
            pl.semaphore_signal(barrier, inc=1, device_id=(nbr,),
                                device_id_type=pl.DeviceIdType.MESH)
        pl.semaphore_wait(barrier, 2)

        for c in range(N_DEV):
            out_ref[pl.ds(c * chunk, chunk), :] = jnp.dot(
                x_ref[pl.ds(c * chunk, chunk), :], w_ref[...],
                preferred_element_type=jnp.float32)

        def cw_rows(idx):
            return pl.ds(idx * chunk, chunk)

        for s in range(N_DEV - 1):
            c_cw = lax.rem(d - s + N_DEV, N_DEV)
            c_ccw = lax.rem(d + s, N_DEV)
            if s > 0:
                out_ref[cw_rows(c_cw), pl.ds(0, half)] = (
                    out_ref[cw_rows(c_cw), pl.ds(0, half)] + cw_ref[s - 1])
                out_ref[cw_rows(c_ccw), pl.ds(half, half)] = (
                    out_ref[cw_rows(c_ccw), pl.ds(half, half)] + ccw_ref[s - 1])
            cw = pltpu.make_async_remote_copy(
                src_ref=out_ref.at[cw_rows(c_cw), pl.ds(0, half)],
                dst_ref=cw_ref.at[s],
                send_sem=cw_send.at[s], recv_sem=cw_recv.at[s],
                device_id=(right,), device_id_type=pl.DeviceIdType.MESH)
            ccw = pltpu.make_async_remote_copy(
                src_ref=out_ref.at[cw_rows(c_ccw), pl.ds(half, half)],
                dst_ref=ccw_ref.at[s],
                send_sem=ccw_send.at[s], recv_sem=ccw_recv.at[s],
                device_id=(left,), device_id_type=pl.DeviceIdType.MESH)
            cw.start()
            ccw.start()
            cw.wait()
            ccw.wait()

        oc_cw = lax.rem(d + 1, N_DEV)
        oc_ccw = lax.rem(d + 3, N_DEV)
        out_ref[cw_rows(oc_cw), pl.ds(0, half)] = jnp.maximum(
            out_ref[cw_rows(oc_cw), pl.ds(0, half)] + cw_ref[N_DEV - 2], 0.0)
        out_ref[cw_rows(oc_ccw), pl.ds(half, half)] = jnp.maximum(
            out_ref[cw_rows(oc_ccw), pl.ds(half, half)] + ccw_ref[N_DEV - 2],
            0.0)

        for t in range(N_DEV - 1):
            slot = (N_DEV - 1) + t
            g_cw = lax.rem(d + 1 - t + N_DEV, N_DEV)
            g_ccw = lax.rem(d + 3 + t, N_DEV)
            cw = pltpu.make_async_remote_copy(
                src_ref=out_ref.at[cw_rows(g_cw), pl.ds(0, half)],
                dst_ref=cw_ref.at[slot],
                send_sem=cw_send.at[slot], recv_sem=cw_recv.at[slot],
                device_id=(right,), device_id_type=pl.DeviceIdType.MESH)
            ccw = pltpu.make_async_remote_copy(
                src_ref=out_ref.at[cw_rows(g_ccw), pl.ds(half, half)],
                dst_ref=ccw_ref.at[slot],
                send_sem=ccw_send.at[slot], recv_sem=ccw_recv.at[slot],
                device_id=(left,), device_id_type=pl.DeviceIdType.MESH)
            cw.start()
            ccw.start()
            cw.wait()
            ccw.wait()
            r_cw = lax.rem(d - t + N_DEV, N_DEV)
            r_ccw = lax.rem(d + t, N_DEV)
            out_ref[cw_rows(r_cw), pl.ds(0, half)] = cw_ref[slot]
            out_ref[cw_rows(r_ccw), pl.ds(half, half)] = ccw_ref[slot]

    n_slots = 2 * (N_DEV - 1)
    return pl.pallas_call(
        body,
        out_shape=jax.ShapeDtypeStruct((m, n), jnp.float32),
        in_specs=[pl.BlockSpec(memory_space=pltpu.VMEM),
                  pl.BlockSpec(memory_space=pltpu.VMEM)],
        out_specs=pl.BlockSpec(memory_space=pltpu.VMEM),
        scratch_shapes=[
            pltpu.VMEM((n_slots, chunk, half), jnp.float32),
            pltpu.VMEM((n_slots, chunk, half), jnp.float32),
            pltpu.SemaphoreType.DMA((n_slots,)),
            pltpu.SemaphoreType.DMA((n_slots,)),
            pltpu.SemaphoreType.DMA((n_slots,)),
            pltpu.SemaphoreType.DMA((n_slots,)),
        ],
        compiler_params=pltpu.CompilerParams(
            collective_id=0, vmem_limit_bytes=100 * 1024 * 1024),
    )(x, w_mat)


# device time: 172636 ns/iter; 1.0243x vs baseline; 1.0243x over previous
import jax
import jax.numpy as jnp
from jax import lax
from jax.experimental import pallas as pl
from jax.experimental.pallas import tpu as pltpu

N_DEV = 4


def kernel(x, w_mat):
    m, k = x.shape
    _, n = w_mat.shape
    chunk = m // N_DEV
    half = n // 2

    def body(x_ref, w_ref, out_ref, cw_ref, ccw_ref,
             cw_send, cw_recv, ccw_send, ccw_recv):
        d = lax.axis_index("i")
        right = lax.rem(d + 1, N_DEV)
        left = lax.rem(d + 3, N_DEV)

        def rows(idx):
            return pl.ds(idx * chunk, chunk)

        cw_cols = pl.ds(0, half)
        ccw_cols = pl.ds(half, half)

        def gemm(c):
            out_ref[rows(c), :] = jnp.dot(
                x_ref[rows(c), :], w_ref[...],
                preferred_element_type=jnp.float32)

        def make_pair(slot, src_cw, src_ccw):
            cw = pltpu.make_async_remote_copy(
                src_ref=src_cw, dst_ref=cw_ref.at[slot],
                send_sem=cw_send.at[slot], recv_sem=cw_recv.at[slot],
                device_id=(right,), device_id_type=pl.DeviceIdType.MESH)
            ccw = pltpu.make_async_remote_copy(
                src_ref=src_ccw, dst_ref=ccw_ref.at[slot],
                send_sem=ccw_send.at[slot], recv_sem=ccw_recv.at[slot],
                device_id=(left,), device_id_type=pl.DeviceIdType.MESH)
            return cw, ccw

        gemm(d)

        barrier = pltpu.get_barrier_semaphore()
        for nbr in (left, right):
            pl.semaphore_signal(barrier, inc=1, device_id=(nbr,),
                                device_id_type=pl.DeviceIdType.MESH)
        pl.semaphore_wait(barrier, 2)

        pend = make_pair(0, out_ref.at[rows(d), cw_cols],
                         out_ref.at[rows(d), ccw_cols])
        pend[0].start()
        pend[1].start()

        gemm(lax.rem(d + 1, N_DEV))
        gemm(lax.rem(d + 3, N_DEV))
        gemm(lax.rem(d + 2, N_DEV))

        for s in range(N_DEV - 1):
            pend[0].wait()
            pend[1].wait()
            c_cw = lax.rem(d - s - 1 + N_DEV, N_DEV)
            c_ccw = lax.rem(d + s + 1, N_DEV)
            if s < N_DEV - 2:
                out_ref[rows(c_cw), cw_cols] = (
                    out_ref[rows(c_cw), cw_cols] + cw_ref[s])
                out_ref[rows(c_ccw), ccw_cols] = (
                    out_ref[rows(c_ccw), ccw_cols] + ccw_ref[s])
                pend = make_pair(s + 1, out_ref.at[rows(c_cw), cw_cols],
                                 out_ref.at[rows(c_ccw), ccw_cols])
                pend[0].start()
                pend[1].start()
            else:
                out_ref[rows(c_cw), cw_cols] = jnp.maximum(
                    out_ref[rows(c_cw), cw_cols] + cw_ref[s], 0.0)
                out_ref[rows(c_ccw), ccw_cols] = jnp.maximum(
                    out_ref[rows(c_ccw), ccw_cols] + ccw_ref[s], 0.0)

        oc_cw = lax.rem(d + 1, N_DEV)
        oc_ccw = lax.rem(d + 3, N_DEV)

        for t in range(N_DEV - 1):
            slot = (N_DEV - 1) + t
            if t == 0:
                src_cw = out_ref.at[rows(oc_cw), cw_cols]
                src_ccw = out_ref.at[rows(oc_ccw), ccw_cols]
            else:
                src_cw = cw_ref.at[slot - 1]
                src_ccw = ccw_ref.at[slot - 1]
            pend = make_pair(slot, src_cw, src_ccw)
            pend[0].start()
            pend[1].start()
            if t > 0:
                r_cw = lax.rem(d - (t - 1) + N_DEV, N_DEV)
                r_ccw = lax.rem(d + t - 1, N_DEV)
                out_ref[rows(r_cw), cw_cols] = cw_ref[slot - 1]
                out_ref[rows(r_ccw), ccw_cols] = ccw_ref[slot - 1]
            pend[0].wait()
            pend[1].wait()
        out_ref[rows(lax.rem(d + 2, N_DEV)), cw_cols] = cw_ref[2 * N_DEV - 3]
        out_ref[rows(lax.rem(d + 2, N_DEV)), ccw_cols] = ccw_ref[2 * N_DEV - 3]

    n_slots = 2 * (N_DEV - 1)
    return pl.pallas_call(
        body,
        out_shape=jax.ShapeDtypeStruct((m, n), jnp.float32),
        in_specs=[pl.BlockSpec(memory_space=pltpu.VMEM),
                  pl.BlockSpec(memory_space=pltpu.VMEM)],
        out_specs=pl.BlockSpec(memory_space=pltpu.VMEM),
        scratch_shapes=[
            pltpu.VMEM((n_slots, chunk, half), jnp.float32),
            pltpu.VMEM((n_slots, chunk, half), jnp.float32),
            pltpu.SemaphoreType.DMA((n_slots,)),
            pltpu.SemaphoreType.DMA((n_slots,)),
            pltpu.SemaphoreType.DMA((n_slots,)),
            pltpu.SemaphoreType.DMA((n_slots,)),
        ],
        compiler_params=pltpu.CompilerParams(
            collective_id=0, vmem_limit_bytes=100 * 1024 * 1024),
    )(x, w_mat)


# device time: 162855 ns/iter; 1.0859x vs baseline; 1.0601x over previous
import jax
import jax.numpy as jnp
from jax import lax
from jax.experimental import pallas as pl
from jax.experimental.pallas import tpu as pltpu

N_DEV = 4
SUBS = 2


def kernel(x, w_mat):
    m, k = x.shape
    _, n = w_mat.shape
    chunk = m // N_DEV
    half = n // 2
    subrows = chunk // SUBS
    n_steps = 2 * (N_DEV - 1)
    n_slots = n_steps * SUBS

    def body(x_ref, w_ref, out_ref, cw_ref, ccw_ref,
             cw_send, cw_recv, ccw_send, ccw_recv):
        d = lax.axis_index("i")
        right = lax.rem(d + 1, N_DEV)
        left = lax.rem(d + 3, N_DEV)

        cw_cols = pl.ds(0, half)
        ccw_cols = pl.ds(half, half)

        def srows(c, j):
            return pl.ds(c * chunk + j * subrows, subrows)

        def slot(step, j):
            return step * SUBS + j

        def mk(step, j, src_cw, src_ccw):
            s_ = slot(step, j)
            cw = pltpu.make_async_remote_copy(
                src_ref=src_cw, dst_ref=cw_ref.at[s_],
                send_sem=cw_send.at[s_], recv_sem=cw_recv.at[s_],
                device_id=(right,), device_id_type=pl.DeviceIdType.MESH)
            ccw = pltpu.make_async_remote_copy(
                src_ref=src_ccw, dst_ref=ccw_ref.at[s_],
                send_sem=ccw_send.at[s_], recv_sem=ccw_recv.at[s_],
                device_id=(left,), device_id_type=pl.DeviceIdType.MESH)
            return cw, ccw

        def start(pair):
            pair[0].start()
            pair[1].start()

        def wait(pair):
            pair[0].wait()
            pair[1].wait()

        def gemm_rows(rs):
            out_ref[rs, :] = jnp.dot(
                x_ref[rs, :], w_ref[...],
                preferred_element_type=jnp.float32)

        gemm_rows(srows(d, 0))

        barrier = pltpu.get_barrier_semaphore()
        for nbr in (left, right):
            pl.semaphore_signal(barrier, inc=1, device_id=(nbr,),
                                device_id_type=pl.DeviceIdType.MESH)
        pl.semaphore_wait(barrier, 2)

        pend = {}
        pend[(0, 0)] = mk(0, 0, out_ref.at[srows(d, 0), cw_cols],
                          out_ref.at[srows(d, 0), ccw_cols])
        start(pend[(0, 0)])
        gemm_rows(srows(d, 1))
        pend[(0, 1)] = mk(0, 1, out_ref.at[srows(d, 1), cw_cols],
                          out_ref.at[srows(d, 1), ccw_cols])
        start(pend[(0, 1)])
        for c_off in (1, 3, 2):
            c = lax.rem(d + c_off, N_DEV)
            gemm_rows(pl.ds(c * chunk, chunk))

        for s in range(N_DEV - 2):
            rc_cw = lax.rem(d - s - 1 + N_DEV, N_DEV)
            rc_ccw = lax.rem(d + s + 1, N_DEV)
            for j in range(SUBS):
                wait(pend.pop((s, j)))
                out_ref[srows(rc_cw, j), cw_cols] = (
                    out_ref[srows(rc_cw, j), cw_cols] + cw_ref[slot(s, j)])
                out_ref[srows(rc_ccw, j), ccw_cols] = (
                    out_ref[srows(rc_ccw, j), ccw_cols] + ccw_ref[slot(s, j)])
                pend[(s + 1, j)] = mk(
                    s + 1, j, out_ref.at[srows(rc_cw, j), cw_cols],
                    out_ref.at[srows(rc_ccw, j), ccw_cols])
                start(pend[(s + 1, j)])

        s = N_DEV - 2
        oc_cw = lax.rem(d + 1, N_DEV)
        oc_ccw = lax.rem(d + 3, N_DEV)
        for j in range(SUBS):
            wait(pend.pop((s, j)))
            out_ref[srows(oc_cw, j), cw_cols] = jnp.maximum(
                out_ref[srows(oc_cw, j), cw_cols] + cw_ref[slot(s, j)], 0.0)
            out_ref[srows(oc_ccw, j), ccw_cols] = jnp.maximum(
                out_ref[srows(oc_ccw, j), ccw_cols] + ccw_ref[slot(s, j)], 0.0)
            pend[(s + 1, j)] = mk(
                s + 1, j, out_ref.at[srows(oc_cw, j), cw_cols],
                out_ref.at[srows(oc_ccw, j), ccw_cols])
            start(pend[(s + 1, j)])

        for t in range(N_DEV - 1):
            step = (N_DEV - 1) + t
            r_cw = lax.rem(d - t + N_DEV, N_DEV)
            r_ccw = lax.rem(d + t, N_DEV)
            for j in range(SUBS):
                wait(pend.pop((step, j)))
                if t < N_DEV - 2:
                    pend[(step + 1, j)] = mk(
                        step + 1, j, cw_ref.at[slot(step, j)],
                        ccw_ref.at[slot(step, j)])
                    start(pend[(step + 1, j)])
                out_ref[srows(r_cw, j), cw_cols] = cw_ref[slot(step, j)]
                out_ref[srows(r_ccw, j), ccw_cols] = ccw_ref[slot(step, j)]

    return pl.pallas_call(
        body,
        out_shape=jax.ShapeDtypeStruct((m, n), jnp.float32),
        in_specs=[pl.BlockSpec(memory_space=pltpu.VMEM),
                  pl.BlockSpec(memory_space=pltpu.VMEM)],
        out_specs=pl.BlockSpec(memory_space=pltpu.VMEM),
        scratch_shapes=[
            pltpu.VMEM((n_slots, subrows, half), jnp.float32),
            pltpu.VMEM((n_slots, subrows, half), jnp.float32),
            pltpu.SemaphoreType.DMA((n_slots,)),
            pltpu.SemaphoreType.DMA((n_slots,)),
            pltpu.SemaphoreType.DMA((n_slots,)),
            pltpu.SemaphoreType.DMA((n_slots,)),
        ],
        compiler_params=pltpu.CompilerParams(
            collective_id=0, vmem_limit_bytes=100 * 1024 * 1024),
    )(x, w_mat)


# device time: 158331 ns/iter; 1.1169x vs baseline; 1.0286x over previous
import jax
import jax.numpy as jnp
from jax import lax
from jax.experimental import pallas as pl
from jax.experimental.pallas import tpu as pltpu

N_DEV = 4
SUBS = 2


def kernel(x, w_mat):
    m, k = x.shape
    _, n = w_mat.shape
    chunk = m // N_DEV
    half = n // 2
    subrows = chunk // SUBS
    n_steps = 2 * (N_DEV - 1)
    n_slots = n_steps * SUBS
    n_dmas = 2 * SUBS * N_DEV

    def body(x_ref, w_ref, out_ref, acc_ref, cw_ref, ccw_ref,
             cw_send, cw_recv, ccw_send, ccw_recv, out_sems):
        d = lax.axis_index("i")
        right = lax.rem(d + 1, N_DEV)
        left = lax.rem(d + 3, N_DEV)

        cw_cols = pl.ds(0, half)
        ccw_cols = pl.ds(half, half)

        def srows(c, j):
            return pl.ds(c * chunk + j * subrows, subrows)

        def slot(step, j):
            return step * SUBS + j

        def mk(step, j, src_cw, src_ccw):
            s_ = slot(step, j)
            cw = pltpu.make_async_remote_copy(
                src_ref=src_cw, dst_ref=cw_ref.at[s_],
                send_sem=cw_send.at[s_], recv_sem=cw_recv.at[s_],
                device_id=(right,), device_id_type=pl.DeviceIdType.MESH)
            ccw = pltpu.make_async_remote_copy(
                src_ref=src_ccw, dst_ref=ccw_ref.at[s_],
                send_sem=ccw_send.at[s_], recv_sem=ccw_recv.at[s_],
                device_id=(left,), device_id_type=pl.DeviceIdType.MESH)
            return cw, ccw

        def start(pair):
            pair[0].start()
            pair[1].start()

        def wait(pair):
            pair[0].wait()
            pair[1].wait()

        out_dmas = []

        def dma_out(src, dst):
            cp = pltpu.make_async_copy(src, dst, out_sems.at[len(out_dmas)])
            cp.start()
            out_dmas.append(cp)

        def gemm_rows(rs):
            acc_ref[rs, :] = jnp.dot(
                x_ref[rs, :], w_ref[...],
                preferred_element_type=jnp.float32)

        gemm_rows(srows(d, 0))

        barrier = pltpu.get_barrier_semaphore()
        for nbr in (left, right):
            pl.semaphore_signal(barrier, inc=1, device_id=(nbr,),
                                device_id_type=pl.DeviceIdType.MESH)
        pl.semaphore_wait(barrier, 2)

        pend = {}
        pend[(0, 0)] = mk(0, 0, acc_ref.at[srows(d, 0), cw_cols],
                          acc_ref.at[srows(d, 0), ccw_cols])
        start(pend[(0, 0)])
        gemm_rows(srows(d, 1))
        pend[(0, 1)] = mk(0, 1, acc_ref.at[srows(d, 1), cw_cols],
                          acc_ref.at[srows(d, 1), ccw_cols])
        start(pend[(0, 1)])
        for c_off in (1, 3, 2):
            c = lax.rem(d + c_off, N_DEV)
            gemm_rows(pl.ds(c * chunk, chunk))

        for s in range(N_DEV - 2):
            rc_cw = lax.rem(d - s - 1 + N_DEV, N_DEV)
            rc_ccw = lax.rem(d + s + 1, N_DEV)
            for j in range(SUBS):
                wait(pend.pop((s, j)))
                acc_ref[srows(rc_cw, j), cw_cols] = (
                    acc_ref[srows(rc_cw, j), cw_cols] + cw_ref[slot(s, j)])
                acc_ref[srows(rc_ccw, j), ccw_cols] = (
                    acc_ref[srows(rc_ccw, j), ccw_cols] + ccw_ref[slot(s, j)])
                pend[(s + 1, j)] = mk(
                    s + 1, j, acc_ref.at[srows(rc_cw, j), cw_cols],
                    acc_ref.at[srows(rc_ccw, j), ccw_cols])
                start(pend[(s + 1, j)])

        s = N_DEV - 2
        oc_cw = lax.rem(d + 1, N_DEV)
        oc_ccw = lax.rem(d + 3, N_DEV)
        for j in range(SUBS):
            wait(pend.pop((s, j)))
            acc_ref[srows(oc_cw, j), cw_cols] = jnp.maximum(
                acc_ref[srows(oc_cw, j), cw_cols] + cw_ref[slot(s, j)], 0.0)
            acc_ref[srows(oc_ccw, j), ccw_cols] = jnp.maximum(
                acc_ref[srows(oc_ccw, j), ccw_cols] + ccw_ref[slot(s, j)], 0.0)
            pend[(s + 1, j)] = mk(
                s + 1, j, acc_ref.at[srows(oc_cw, j), cw_cols],
                acc_ref.at[srows(oc_ccw, j), ccw_cols])
            start(pend[(s + 1, j)])
            dma_out(acc_ref.at[srows(oc_cw, j), cw_cols],
                    out_ref.at[srows(oc_cw, j), cw_cols])
            dma_out(acc_ref.at[srows(oc_ccw, j), ccw_cols],
                    out_ref.at[srows(oc_ccw, j), ccw_cols])

        for t in range(N_DEV - 1):
            step = (N_DEV - 1) + t
            r_cw = lax.rem(d - t + N_DEV, N_DEV)
            r_ccw = lax.rem(d + t, N_DEV)
            for j in range(SUBS):
                wait(pend.pop((step, j)))
                if t < N_DEV - 2:
                    pend[(step + 1, j)] = mk(
                        step + 1, j, cw_ref.at[slot(step, j)],
                        ccw_ref.at[slot(step, j)])
                    start(pend[(step + 1, j)])
                dma_out(cw_ref.at[slot(step, j)],
                        out_ref.at[srows(r_cw, j), cw_cols])
                dma_out(ccw_ref.at[slot(step, j)],
                        out_ref.at[srows(r_ccw, j), ccw_cols])

        for cp in out_dmas:
            cp.wait()

    return pl.pallas_call(
        body,
        out_shape=jax.ShapeDtypeStruct((m, n), jnp.float32),
        in_specs=[pl.BlockSpec(memory_space=pltpu.VMEM),
                  pl.BlockSpec(memory_space=pltpu.VMEM)],
        out_specs=pl.BlockSpec(memory_space=pl.ANY),
        scratch_shapes=[
            pltpu.VMEM((m, n), jnp.float32),
            pltpu.VMEM((n_slots, subrows, half), jnp.float32),
            pltpu.VMEM((n_slots, subrows, half), jnp.float32),
            pltpu.SemaphoreType.DMA((n_slots,)),
            pltpu.SemaphoreType.DMA((n_slots,)),
            pltpu.SemaphoreType.DMA((n_slots,)),
            pltpu.SemaphoreType.DMA((n_slots,)),
            pltpu.SemaphoreType.DMA((n_dmas,)),
        ],
        compiler_params=pltpu.CompilerParams(
            collective_id=0, vmem_limit_bytes=100 * 1024 * 1024),
    )(x, w_mat)


# device time: 158324 ns/iter; 1.1169x vs baseline; 1.0000x over previous
import jax
import jax.numpy as jnp
from jax import lax
from jax.experimental import pallas as pl
from jax.experimental.pallas import tpu as pltpu

N_DEV = 4
SUBS = 4


def kernel(x, w_mat):
    m, k = x.shape
    _, n = w_mat.shape
    chunk = m // N_DEV
    half = n // 2
    subrows = chunk // SUBS
    n_steps = 2 * (N_DEV - 1)
    n_slots = n_steps * SUBS
    n_dmas = 2 * SUBS * N_DEV

    def body(x_ref, w_ref, out_ref, acc_ref, cw_ref, ccw_ref,
             cw_send, cw_recv, ccw_send, ccw_recv, out_sems):
        d = lax.axis_index("i")
        right = lax.rem(d + 1, N_DEV)
        left = lax.rem(d + 3, N_DEV)

        cw_cols = pl.ds(0, half)
        ccw_cols = pl.ds(half, half)

        def srows(c, j):
            return pl.ds(c * chunk + j * subrows, subrows)

        def slot(step, j):
            return step * SUBS + j

        def mk(step, j, src_cw, src_ccw):
            s_ = slot(step, j)
            cw = pltpu.make_async_remote_copy(
                src_ref=src_cw, dst_ref=cw_ref.at[s_],
                send_sem=cw_send.at[s_], recv_sem=cw_recv.at[s_],
                device_id=(right,), device_id_type=pl.DeviceIdType.MESH)
            ccw = pltpu.make_async_remote_copy(
                src_ref=src_ccw, dst_ref=ccw_ref.at[s_],
                send_sem=ccw_send.at[s_], recv_sem=ccw_recv.at[s_],
                device_id=(left,), device_id_type=pl.DeviceIdType.MESH)
            return cw, ccw

        def start(pair):
            pair[0].start()
            pair[1].start()

        def wait(pair):
            pair[0].wait()
            pair[1].wait()

        out_dmas = []

        def dma_out(src, dst):
            cp = pltpu.make_async_copy(src, dst, out_sems.at[len(out_dmas)])
            cp.start()
            out_dmas.append(cp)

        def gemm_rows(rs):
            acc_ref[rs, :] = jnp.dot(
                x_ref[rs, :], w_ref[...],
                preferred_element_type=jnp.float32)

        gemm_rows(srows(d, 0))

        barrier = pltpu.get_barrier_semaphore()
        for nbr in (left, right):
            pl.semaphore_signal(barrier, inc=1, device_id=(nbr,),
                                device_id_type=pl.DeviceIdType.MESH)
        pl.semaphore_wait(barrier, 2)

        pend = {}
        pend[(0, 0)] = mk(0, 0, acc_ref.at[srows(d, 0), cw_cols],
                          acc_ref.at[srows(d, 0), ccw_cols])
        start(pend[(0, 0)])
        for j in range(1, SUBS):
            gemm_rows(srows(d, j))
            pend[(0, j)] = mk(0, j, acc_ref.at[srows(d, j), cw_cols],
                              acc_ref.at[srows(d, j), ccw_cols])
            start(pend[(0, j)])
        for c_off in (1, 3, 2):
            c = lax.rem(d + c_off, N_DEV)
            gemm_rows(pl.ds(c * chunk, chunk))

        for s in range(N_DEV - 2):
            rc_cw = lax.rem(d - s - 1 + N_DEV, N_DEV)
            rc_ccw = lax.rem(d + s + 1, N_DEV)
            for j in range(SUBS):
                wait(pend.pop((s, j)))
                acc_ref[srows(rc_cw, j), cw_cols] = (
                    acc_ref[srows(rc_cw, j), cw_cols] + cw_ref[slot(s, j)])
                acc_ref[srows(rc_ccw, j), ccw_cols] = (
                    acc_ref[srows(rc_ccw, j), ccw_cols] + ccw_ref[slot(s, j)])
                pend[(s + 1, j)] = mk(
                    s + 1, j, acc_ref.at[srows(rc_cw, j), cw_cols],
                    acc_ref.at[srows(rc_ccw, j), ccw_cols])
                start(pend[(s + 1, j)])

        s = N_DEV - 2
        oc_cw = lax.rem(d + 1, N_DEV)
        oc_ccw = lax.rem(d + 3, N_DEV)
        for j in range(SUBS):
            wait(pend.pop((s, j)))
            acc_ref[srows(oc_cw, j), cw_cols] = jnp.maximum(
                acc_ref[srows(oc_cw, j), cw_cols] + cw_ref[slot(s, j)], 0.0)
            acc_ref[srows(oc_ccw, j), ccw_cols] = jnp.maximum(
                acc_ref[srows(oc_ccw, j), ccw_cols] + ccw_ref[slot(s, j)], 0.0)
            pend[(s + 1, j)] = mk(
                s + 1, j, acc_ref.at[srows(oc_cw, j), cw_cols],
                acc_ref.at[srows(oc_ccw, j), ccw_cols])
            start(pend[(s + 1, j)])
            dma_out(acc_ref.at[srows(oc_cw, j), cw_cols],
                    out_ref.at[srows(oc_cw, j), cw_cols])
            dma_out(acc_ref.at[srows(oc_ccw, j), ccw_cols],
                    out_ref.at[srows(oc_ccw, j), ccw_cols])

        for t in range(N_DEV - 1):
            step = (N_DEV - 1) + t
            r_cw = lax.rem(d - t + N_DEV, N_DEV)
            r_ccw = lax.rem(d + t, N_DEV)
            for j in range(SUBS):
                wait(pend.pop((step, j)))
                if t < N_DEV - 2:
                    pend[(step + 1, j)] = mk(
                        step + 1, j, cw_ref.at[slot(step, j)],
                        ccw_ref.at[slot(step, j)])
                    start(pend[(step + 1, j)])
                dma_out(cw_ref.at[slot(step, j)],
                        out_ref.at[srows(r_cw, j), cw_cols])
                dma_out(ccw_ref.at[slot(step, j)],
                        out_ref.at[srows(r_ccw, j), ccw_cols])

        for cp in out_dmas:
            cp.wait()

    return pl.pallas_call(
        body,
        out_shape=jax.ShapeDtypeStruct((m, n), jnp.float32),
        in_specs=[pl.BlockSpec(memory_space=pltpu.VMEM),
                  pl.BlockSpec(memory_space=pltpu.VMEM)],
        out_specs=pl.BlockSpec(memory_space=pl.ANY),
        scratch_shapes=[
            pltpu.VMEM((m, n), jnp.float32),
            pltpu.VMEM((n_slots, subrows, half), jnp.float32),
            pltpu.VMEM((n_slots, subrows, half), jnp.float32),
            pltpu.SemaphoreType.DMA((n_slots,)),
            pltpu.SemaphoreType.DMA((n_slots,)),
            pltpu.SemaphoreType.DMA((n_slots,)),
            pltpu.SemaphoreType.DMA((n_slots,)),
            pltpu.SemaphoreType.DMA((n_dmas,)),
        ],
        compiler_params=pltpu.CompilerParams(
            collective_id=0, vmem_limit_bytes=100 * 1024 * 1024),
    )(x, w_mat)


# device time: 157592 ns/iter; 1.1221x vs baseline; 1.0046x over previous
import jax
import jax.numpy as jnp
from jax import lax
from jax.experimental import pallas as pl
from jax.experimental.pallas import tpu as pltpu

N_DEV = 4
SUBS = 2


def kernel(x, w_mat):
    m, k = x.shape
    _, n = w_mat.shape
    chunk = m // N_DEV
    half = n // 2
    subrows = chunk // SUBS
    n_steps = 2 * (N_DEV - 1)
    n_slots = n_steps * SUBS
    n_dmas = 2 * SUBS * N_DEV

    def body(x_ref, w_ref, out_ref, acc_ref, xv_ref, wv_ref, cw_ref, ccw_ref,
             cw_send, cw_recv, ccw_send, ccw_recv, out_sems, in_sems):
        d = lax.axis_index("i")
        right = lax.rem(d + 1, N_DEV)
        left = lax.rem(d + 3, N_DEV)

        cw_cols = pl.ds(0, half)
        ccw_cols = pl.ds(half, half)

        def srows(c, j):
            return pl.ds(c * chunk + j * subrows, subrows)

        def slot(step, j):
            return step * SUBS + j

        def mk(step, j, src_cw, src_ccw):
            s_ = slot(step, j)
            cw = pltpu.make_async_remote_copy(
                src_ref=src_cw, dst_ref=cw_ref.at[s_],
                send_sem=cw_send.at[s_], recv_sem=cw_recv.at[s_],
                device_id=(right,), device_id_type=pl.DeviceIdType.MESH)
            ccw = pltpu.make_async_remote_copy(
                src_ref=src_ccw, dst_ref=ccw_ref.at[s_],
                send_sem=ccw_send.at[s_], recv_sem=ccw_recv.at[s_],
                device_id=(left,), device_id_type=pl.DeviceIdType.MESH)
            return cw, ccw

        def start(pair):
            pair[0].start()
            pair[1].start()

        def wait(pair):
            pair[0].wait()
            pair[1].wait()

        out_dmas = []

        def dma_out(src, dst):
            cp = pltpu.make_async_copy(src, dst, out_sems.at[len(out_dmas)])
            cp.start()
            out_dmas.append(cp)

        def gemm_rows(rs):
            acc_ref[rs, :] = jnp.dot(
                xv_ref[rs, :], wv_ref[...],
                preferred_element_type=jnp.float32)

        w_cp = pltpu.make_async_copy(w_ref, wv_ref, in_sems.at[0])
        w_cp.start()
        x_cps = []
        for i, c_off in enumerate((0, 1, 3, 2)):
            c = lax.rem(d + c_off, N_DEV)
            cp = pltpu.make_async_copy(x_ref.at[pl.ds(c * chunk, chunk), :],
                                       xv_ref.at[pl.ds(c * chunk, chunk), :],
                                       in_sems.at[1 + i])
            cp.start()
            x_cps.append(cp)

        w_cp.wait()
        x_cps[0].wait()
        gemm_rows(srows(d, 0))

        barrier = pltpu.get_barrier_semaphore()
        for nbr in (left, right):
            pl.semaphore_signal(barrier, inc=1, device_id=(nbr,),
                                device_id_type=pl.DeviceIdType.MESH)
        pl.semaphore_wait(barrier, 2)

        pend = {}
        pend[(0, 0)] = mk(0, 0, acc_ref.at[srows(d, 0), cw_cols],
                          acc_ref.at[srows(d, 0), ccw_cols])
        start(pend[(0, 0)])
        for j in range(1, SUBS):
            gemm_rows(srows(d, j))
            pend[(0, j)] = mk(0, j, acc_ref.at[srows(d, j), cw_cols],
                              acc_ref.at[srows(d, j), ccw_cols])
            start(pend[(0, j)])
        for i, c_off in enumerate((1, 3, 2)):
            c = lax.rem(d + c_off, N_DEV)
            x_cps[1 + i].wait()
            gemm_rows(pl.ds(c * chunk, chunk))

        for s in range(N_DEV - 2):
            rc_cw = lax.rem(d - s - 1 + N_DEV, N_DEV)
            rc_ccw = lax.rem(d + s + 1, N_DEV)
            for j in range(SUBS):
                wait(pend.pop((s, j)))
                acc_ref[srows(rc_cw, j), cw_cols] = (
                    acc_ref[srows(rc_cw, j), cw_cols] + cw_ref[slot(s, j)])
                acc_ref[srows(rc_ccw, j), ccw_cols] = (
                    acc_ref[srows(rc_ccw, j), ccw_cols] + ccw_ref[slot(s, j)])
                pend[(s + 1, j)] = mk(
                    s + 1, j, acc_ref.at[srows(rc_cw, j), cw_cols],
                    acc_ref.at[srows(rc_ccw, j), ccw_cols])
                start(pend[(s + 1, j)])

        s = N_DEV - 2
        oc_cw = lax.rem(d + 1, N_DEV)
        oc_ccw = lax.rem(d + 3, N_DEV)
        for j in range(SUBS):
            wait(pend.pop((s, j)))
            acc_ref[srows(oc_cw, j), cw_cols] = jnp.maximum(
                acc_ref[srows(oc_cw, j), cw_cols] + cw_ref[slot(s, j)], 0.0)
            acc_ref[srows(oc_ccw, j), ccw_cols] = jnp.maximum(
                acc_ref[srows(oc_ccw, j), ccw_cols] + ccw_ref[slot(s, j)], 0.0)
            pend[(s + 1, j)] = mk(
                s + 1, j, acc_ref.at[srows(oc_cw, j), cw_cols],
                acc_ref.at[srows(oc_ccw, j), ccw_cols])
            start(pend[(s + 1, j)])
            dma_out(acc_ref.at[srows(oc_cw, j), cw_cols],
                    out_ref.at[srows(oc_cw, j), cw_cols])
            dma_out(acc_ref.at[srows(oc_ccw, j), ccw_cols],
                    out_ref.at[srows(oc_ccw, j), ccw_cols])

        for t in range(N_DEV - 1):
            step = (N_DEV - 1) + t
            r_cw = lax.rem(d - t + N_DEV, N_DEV)
            r_ccw = lax.rem(d + t, N_DEV)
            for j in range(SUBS):
                wait(pend.pop((step, j)))
                if t < N_DEV - 2:
                    pend[(step + 1, j)] = mk(
                        step + 1, j, cw_ref.at[slot(step, j)],
                        ccw_ref.at[slot(step, j)])
                    start(pend[(step + 1, j)])
                dma_out(cw_ref.at[slot(step, j)],
                        out_ref.at[srows(r_cw, j), cw_cols])
                dma_out(ccw_ref.at[slot(step, j)],
                        out_ref.at[srows(r_ccw, j), ccw_cols])

        for cp in out_dmas:
            cp.wait()

    return pl.pallas_call(
        body,
        out_shape=jax.ShapeDtypeStruct((m, n), jnp.float32),
        in_specs=[pl.BlockSpec(memory_space=pl.ANY),
                  pl.BlockSpec(memory_space=pl.ANY)],
        out_specs=pl.BlockSpec(memory_space=pl.ANY),
        scratch_shapes=[
            pltpu.VMEM((m, n), jnp.float32),
            pltpu.VMEM((m, k), jnp.float32),
            pltpu.VMEM((k, n), jnp.float32),
            pltpu.VMEM((n_slots, subrows, half), jnp.float32),
            pltpu.VMEM((n_slots, subrows, half), jnp.float32),
            pltpu.SemaphoreType.DMA((n_slots,)),
            pltpu.SemaphoreType.DMA((n_slots,)),
            pltpu.SemaphoreType.DMA((n_slots,)),
            pltpu.SemaphoreType.DMA((n_slots,)),
            pltpu.SemaphoreType.DMA((n_dmas,)),
            pltpu.SemaphoreType.DMA((5,)),
        ],
        compiler_params=pltpu.CompilerParams(
            collective_id=0, vmem_limit_bytes=100 * 1024 * 1024),
    )(x, w_mat)


# device time: 156516 ns/iter; 1.1298x vs baseline; 1.0069x over previous
import jax
import jax.numpy as jnp
from jax import lax
from jax.experimental import pallas as pl
from jax.experimental.pallas import tpu as pltpu

N_DEV = 4
SUBS = 2


def kernel(x, w_mat):
    m, k = x.shape
    _, n = w_mat.shape
    chunk = m // N_DEV
    half = n // 2
    subrows = chunk // SUBS
    n_steps = 2 * (N_DEV - 1)
    n_slots = n_steps * SUBS
    n_dmas = 2 * SUBS * N_DEV

    def body(x_ref, w_ref, out_ref, acc_ref, xv_ref, wv_ref, cw_ref, ccw_ref,
             cw_send, cw_recv, ccw_send, ccw_recv, out_sems, in_sems):
        d = lax.axis_index("i")
        right = lax.rem(d + 1, N_DEV)
        left = lax.rem(d + 3, N_DEV)

        cw_cols = pl.ds(0, half)
        ccw_cols = pl.ds(half, half)

        def srows(c, j):
            return pl.ds(c * chunk + j * subrows, subrows)

        def slot(step, j):
            return step * SUBS + j

        def mk(step, j, src_cw, src_ccw, cw_tgt=None, ccw_tgt=None):
            s_ = slot(step, j)
            cw = pltpu.make_async_remote_copy(
                src_ref=src_cw, dst_ref=cw_ref.at[s_],
                send_sem=cw_send.at[s_], recv_sem=cw_recv.at[s_],
                device_id=(right if cw_tgt is None else cw_tgt,),
                device_id_type=pl.DeviceIdType.MESH)
            ccw = pltpu.make_async_remote_copy(
                src_ref=src_ccw, dst_ref=ccw_ref.at[s_],
                send_sem=ccw_send.at[s_], recv_sem=ccw_recv.at[s_],
                device_id=(left if ccw_tgt is None else ccw_tgt,),
                device_id_type=pl.DeviceIdType.MESH)
            return cw, ccw

        def start(pair):
            pair[0].start()
            pair[1].start()

        def wait(pair):
            pair[0].wait()
            pair[1].wait()

        out_dmas = []

        def dma_out(src, dst):
            cp = pltpu.make_async_copy(src, dst, out_sems.at[len(out_dmas)])
            cp.start()
            out_dmas.append(cp)

        def gemm_rows(rs):
            acc_ref[rs, :] = jnp.dot(
                xv_ref[rs, :], wv_ref[...],
                preferred_element_type=jnp.float32)

        w_cp = pltpu.make_async_copy(w_ref, wv_ref, in_sems.at[0])
        w_cp.start()
        x_cps = []
        for i, c_off in enumerate((0, 1, 3, 2)):
            c = lax.rem(d + c_off, N_DEV)
            cp = pltpu.make_async_copy(x_ref.at[pl.ds(c * chunk, chunk), :],
                                       xv_ref.at[pl.ds(c * chunk, chunk), :],
                                       in_sems.at[1 + i])
            cp.start()
            x_cps.append(cp)

        w_cp.wait()
        x_cps[0].wait()
        gemm_rows(srows(d, 0))

        barrier = pltpu.get_barrier_semaphore()
        for nbr in (left, right):
            pl.semaphore_signal(barrier, inc=1, device_id=(nbr,),
                                device_id_type=pl.DeviceIdType.MESH)
        pl.semaphore_wait(barrier, 2)

        pend = {}
        pend[(0, 0)] = mk(0, 0, acc_ref.at[srows(d, 0), cw_cols],
                          acc_ref.at[srows(d, 0), ccw_cols])
        start(pend[(0, 0)])
        for j in range(1, SUBS):
            gemm_rows(srows(d, j))
            pend[(0, j)] = mk(0, j, acc_ref.at[srows(d, j), cw_cols],
                              acc_ref.at[srows(d, j), ccw_cols])
            start(pend[(0, j)])
        for i, c_off in enumerate((1, 3, 2)):
            c = lax.rem(d + c_off, N_DEV)
            x_cps[1 + i].wait()
            gemm_rows(pl.ds(c * chunk, chunk))

        for s in range(N_DEV - 2):
            rc_cw = lax.rem(d - s - 1 + N_DEV, N_DEV)
            rc_ccw = lax.rem(d + s + 1, N_DEV)
            for j in range(SUBS):
                wait(pend.pop((s, j)))
                acc_ref[srows(rc_cw, j), cw_cols] = (
                    acc_ref[srows(rc_cw, j), cw_cols] + cw_ref[slot(s, j)])
                acc_ref[srows(rc_ccw, j), ccw_cols] = (
                    acc_ref[srows(rc_ccw, j), ccw_cols] + ccw_ref[slot(s, j)])
                pend[(s + 1, j)] = mk(
                    s + 1, j, acc_ref.at[srows(rc_cw, j), cw_cols],
                    acc_ref.at[srows(rc_ccw, j), ccw_cols])
                start(pend[(s + 1, j)])

        s = N_DEV - 2
        oc_cw = lax.rem(d + 1, N_DEV)
        oc_ccw = lax.rem(d + 3, N_DEV)
        for j in range(SUBS):
            wait(pend.pop((s, j)))
            acc_ref[srows(oc_cw, j), cw_cols] = jnp.maximum(
                acc_ref[srows(oc_cw, j), cw_cols] + cw_ref[slot(s, j)], 0.0)
            acc_ref[srows(oc_ccw, j), ccw_cols] = jnp.maximum(
                acc_ref[srows(oc_ccw, j), ccw_cols] + ccw_ref[slot(s, j)], 0.0)
            pend[(3, j)] = mk(3, j, acc_ref.at[srows(oc_cw, j), cw_cols],
                              acc_ref.at[srows(oc_ccw, j), ccw_cols])
            start(pend[(3, j)])
            pend[(4, j)] = mk(4, j, acc_ref.at[srows(oc_cw, j), cw_cols],
                              acc_ref.at[srows(oc_ccw, j), ccw_cols],
                              cw_tgt=left, ccw_tgt=right)
            start(pend[(4, j)])
            dma_out(acc_ref.at[srows(oc_cw, j), cw_cols],
                    out_ref.at[srows(oc_cw, j), cw_cols])
            dma_out(acc_ref.at[srows(oc_ccw, j), ccw_cols],
                    out_ref.at[srows(oc_ccw, j), ccw_cols])

        r_direct = d
        r_far = lax.rem(d + 2, N_DEV)
        r_fwd_cw = lax.rem(d + 3, N_DEV)
        r_fwd_ccw = lax.rem(d + 1, N_DEV)
        for j in range(SUBS):
            wait(pend.pop((3, j)))
            pend[(5, j)] = mk(5, j, cw_ref.at[slot(3, j)],
                              ccw_ref.at[slot(3, j)])
            start(pend[(5, j)])
            dma_out(cw_ref.at[slot(3, j)],
                    out_ref.at[srows(r_direct, j), cw_cols])
            dma_out(ccw_ref.at[slot(3, j)],
                    out_ref.at[srows(r_direct, j), ccw_cols])
        for j in range(SUBS):
            wait(pend.pop((4, j)))
            dma_out(cw_ref.at[slot(4, j)],
                    out_ref.at[srows(r_far, j), cw_cols])
            dma_out(ccw_ref.at[slot(4, j)],
                    out_ref.at[srows(r_far, j), ccw_cols])
        for j in range(SUBS):
            wait(pend.pop((5, j)))
            dma_out(cw_ref.at[slot(5, j)],
                    out_ref.at[srows(r_fwd_cw, j), cw_cols])
            dma_out(ccw_ref.at[slot(5, j)],
                    out_ref.at[srows(r_fwd_ccw, j), ccw_cols])

        for cp in out_dmas:
            cp.wait()

    return pl.pallas_call(
        body,
        out_shape=jax.ShapeDtypeStruct((m, n), jnp.float32),
        in_specs=[pl.BlockSpec(memory_space=pl.ANY),
                  pl.BlockSpec(memory_space=pl.ANY)],
        out_specs=pl.BlockSpec(memory_space=pl.ANY),
        scratch_shapes=[
            pltpu.VMEM((m, n), jnp.float32),
            pltpu.VMEM((m, k), jnp.float32),
            pltpu.VMEM((k, n), jnp.float32),
            pltpu.VMEM((n_slots, subrows, half), jnp.float32),
            pltpu.VMEM((n_slots, subrows, half), jnp.float32),
            pltpu.SemaphoreType.DMA((n_slots,)),
            pltpu.SemaphoreType.DMA((n_slots,)),
            pltpu.SemaphoreType.DMA((n_slots,)),
            pltpu.SemaphoreType.DMA((n_slots,)),
            pltpu.SemaphoreType.DMA((n_dmas,)),
            pltpu.SemaphoreType.DMA((5,)),
        ],
        compiler_params=pltpu.CompilerParams(
            collective_id=0, vmem_limit_bytes=100 * 1024 * 1024),
    )(x, w_mat)
